# baseline (speedup 1.0000x reference)
import functools

import jax
import jax.numpy as jnp
from jax import lax
from jax.experimental import pallas as pl
from jax.experimental.pallas import tpu as pltpu
from jax.experimental.pallas import tpu_sc as plsc

N_NODES = 10000
N_EDGES = 320000
D = 128

BN = 1000
BE = 2000

NC = 2
NS = 16
NW = NC * NS
EW = N_EDGES // NW
GB = 80
NCH = EW // GB
NP = 10240
SR = NP // NS
NBLK = N_EDGES // BE
BPW = EW // BE
CPB = BE // GB


def _gn0(x, g, b):
    v = jnp.mean(x * x, axis=-1, keepdims=True)
    return x * lax.rsqrt(v + 1e-5) * g + b


def _gnm(x, g, b):
    m = jnp.mean(x, axis=-1, keepdims=True)
    xc = x - m
    v = jnp.mean(xc * xc, axis=-1, keepdims=True)
    return xc * lax.rsqrt(v + 1e-5) * g + b


def _node_body(agts_ref, ctx_ref, qw_ref, qg_ref, qb_ref, w1q_ref, w1c_ref,
               agtw_ref, qwo_ref, cco_ref, a0_ref):
    agts = agts_ref[...]
    q = jnp.maximum(_gn0(jnp.dot(agts, qw_ref[...],
                                preferred_element_type=jnp.float32),
                        qg_ref[...], qb_ref[...]), 0.0)
    qwo_ref[...] = jnp.dot(q, w1q_ref[...], preferred_element_type=jnp.float32)
    cco_ref[...] = jnp.dot(ctx_ref[...], w1c_ref[...],
                           preferred_element_type=jnp.float32)
    a0_ref[...] = jnp.dot(agts, agtw_ref[...],
                          preferred_element_type=jnp.float32)


def _edge_body(g_ref, dt_ref, w1p_ref, b1_ref, dw2_ref,
               g2_ref, b2_ref, w1d_ref, g1_ref, b1c_ref, cw2_ref, c_ref):
    d = jnp.maximum(
        lax.dot_general(dt_ref[0], w1p_ref[...], (((0,), (0,)), ((), ())),
                        preferred_element_type=jnp.float32) + b1_ref[...], 0.0)
    t = jnp.maximum(
        _gn0(jnp.dot(d, dw2_ref[...], preferred_element_type=jnp.float32),
            g2_ref[...], b2_ref[...]), 0.0)
    e = jnp.dot(t, w1d_ref[...], preferred_element_type=jnp.float32)
    e = e + g_ref[...]
    e = jnp.maximum(_gn0(e, g1_ref[...], b1c_ref[...]), 0.0)
    c_ref[...] = jnp.dot(e, cw2_ref[...], preferred_element_type=jnp.float32)


def _epi_body(a0_ref, p0_ref, p1_ref, res_ref, linw_ref, ng_ref, nb_ref,
              lg_ref, lb_ref, out_ref):
    a = a0_ref[...] + p0_ref[0] + p1_ref[0]
    a = jnp.maximum(_gnm(a, ng_ref[...], nb_ref[...]), 0.0)
    a = _gn0(jnp.dot(a, linw_ref[...], preferred_element_type=jnp.float32),
            lg_ref[...], lb_ref[...])
    out_ref[...] = jnp.maximum(a + res_ref[...], 0.0)


def _full(shape):
    return pl.BlockSpec(shape, lambda i: (0,) * len(shape))


def _rows(bs, w):
    return pl.BlockSpec((bs, w), lambda i: (i, 0))


def _node_stage(agts, ctx, query_w, query_g, query_b, w1q, w1c, agt_w):
    f = jax.ShapeDtypeStruct((N_NODES, D), jnp.float32)
    return pl.pallas_call(
        _node_body,
        grid=(N_NODES // BN,),
        in_specs=[_rows(BN, D), _rows(BN, D), _full((D, D)), _full((1, D)),
                  _full((1, D)), _full((D, D)), _full((D, D)), _full((D, D))],
        out_specs=[_rows(BN, D)] * 3,
        out_shape=[f, f, f],
    )(agts, ctx, query_w, query_g, query_b, w1q, w1c, agt_w)


def _edge_stage(g, dt, w1p, b1, dw2, g2, b2, w1d, g1, b1c, cw2):
    return pl.pallas_call(
        _edge_body,
        grid=(N_EDGES // BE,),
        in_specs=[_rows(BE, D),
                  pl.BlockSpec((1, 8, BE), lambda i: (i, 0, 0)),
                  _full((8, D)), _full((1, D)), _full((D, D)), _full((1, D)),
                  _full((1, D)), _full((D, D)), _full((1, D)), _full((1, D)),
                  _full((D, D))],
        out_specs=_rows(BE, D),
        out_shape=jax.ShapeDtypeStruct((N_EDGES, D), jnp.float32),
    )(g, dt, w1p, b1, dw2, g2, b2, w1d, g1, b1c, cw2)


def _epilogue(a0, part, res, lin_w, ng, nb, lg, lb):
    return pl.pallas_call(
        _epi_body,
        grid=(N_NODES // BN,),
        in_specs=[_rows(BN, D),
                  pl.BlockSpec((1, BN, D), lambda i: (0, i, 0)),
                  pl.BlockSpec((1, BN, D), lambda i: (1, i, 0)),
                  _rows(BN, D), _full((D, D))] + [_full((1, D))] * 4,
        out_specs=_rows(BN, D),
        out_shape=jax.ShapeDtypeStruct((N_NODES, D), jnp.float32),
    )(a0, part, part, res, lin_w, ng, nb, lg, lb)


def _sc_gather(qw_tab, cc_tab, ax, ay, cx, cy, hi2d, wi2d):
    f = jnp.float32
    mesh = plsc.VectorSubcoreMesh(core_axis_name="c", subcore_axis_name="s")

    @functools.partial(
        pl.kernel, mesh=mesh,
        compiler_params=pltpu.CompilerParams(needs_layout_passes=False),
        out_type=[jax.ShapeDtypeStruct((N_EDGES, D), f),
                  jax.ShapeDtypeStruct((NBLK, 8, BE), f)],
        scratch_types=[pltpu.VMEM((EW,), jnp.int32),
                       pltpu.VMEM((EW,), jnp.int32),
                       pltpu.VMEM((N_NODES,), f), pltpu.VMEM((N_NODES,), f),
                       pltpu.VMEM((N_NODES,), f), pltpu.VMEM((N_NODES,), f),
                       [pltpu.VMEM((GB, D), f)] * 2,
                       [pltpu.VMEM((GB, D), f)] * 2,
                       pltpu.VMEM((8, BE), f),
                       [pltpu.SemaphoreType.DMA] * 2,
                       [pltpu.SemaphoreType.DMA] * 2,
                       [pltpu.SemaphoreType.DMA] * 2],
    )
    def k(qw_hbm, cc_hbm, ax_hbm, ay_hbm, cx_hbm, cy_hbm, hi_hbm, wi_hbm,
          g_hbm, dt_hbm,
          hi_v, wi_v, ax_v, ay_v, cx_v, cy_v, bq, bc, dtb, sq, sc_, sw):
        wid = lax.axis_index("s") * NC + lax.axis_index("c")
        base = wid * EW
        pltpu.sync_copy(hi_hbm.at[wid], hi_v)
        pltpu.sync_copy(wi_hbm.at[wid], wi_v)
        pltpu.sync_copy(ax_hbm, ax_v)
        pltpu.sync_copy(ay_hbm, ay_v)
        pltpu.sync_copy(cx_hbm, cx_v)
        pltpu.sync_copy(cy_hbm, cy_v)

        zero = jnp.zeros((16,), f)

        def zrow(q, carry):
            dtb[2 + q // (BE // 16), pl.ds((q % (BE // 16)) * 16, 16)] = zero
            return carry

        lax.fori_loop(0, 6 * (BE // 16), zrow, 0)

        def gather_start(j, b):
            pltpu.async_copy(qw_hbm.at[hi_v.at[pl.ds(j * GB, GB)]],
                             bq[b], sq[b])
            pltpu.async_copy(cc_hbm.at[wi_v.at[pl.ds(j * GB, GB)]],
                             bc[b], sc_[b])

        def process(j, b):
            pltpu.make_async_copy(qw_hbm.at[pl.ds(0, GB)], bq[b],
                                  sq[b]).wait()
            pltpu.make_async_copy(cc_hbm.at[pl.ds(0, GB)], bc[b],
                                  sc_[b]).wait()

            def addrow(rr, carry):
                for gi in range(D // 16):
                    s = pl.ds(gi * 16, 16)
                    bq[b][rr, s] = bq[b][rr, s] + bc[b][rr, s]
                return carry

            lax.fori_loop(0, GB, addrow, 0)

            off = (j % CPB) * GB
            for g in range(GB // 16):
                h = hi_v[pl.ds(j * GB + g * 16, 16)]
                w = wi_v[pl.ds(j * GB + g * 16, 16)]
                dtb[0, pl.ds(off + g * 16, 16)] = (
                    plsc.load_gather(ax_v, [h]) - plsc.load_gather(cx_v, [w]))
                dtb[1, pl.ds(off + g * 16, 16)] = (
                    plsc.load_gather(ay_v, [h]) - plsc.load_gather(cy_v, [w]))

            pltpu.async_copy(bq[b], g_hbm.at[pl.ds(base + j * GB, GB)], sw[b])
            pltpu.make_async_copy(bq[b], g_hbm.at[pl.ds(0, GB)], sw[b]).wait()

            @pl.when(j + 2 < NCH)
            def _refill():
                gather_start(j + 2, b)

            @pl.when(j % CPB == CPB - 1)
            def _flush():
                pltpu.sync_copy(dtb, dt_hbm.at[wid * BPW + j // CPB])

        gather_start(0, 0)
        gather_start(1, 1)

        def pair(i, carry):
            process(2 * i, 0)
            process(2 * i + 1, 1)
            return carry

        lax.fori_loop(0, NCH // 2, pair, 0)
        process(NCH - 1, 0)

    return k(qw_tab, cc_tab, ax, ay, cx, cy, hi2d, wi2d)


def _sc_scatter(c_arr, hi2d):
    f = jnp.float32
    mesh = plsc.VectorSubcoreMesh(core_axis_name="c", subcore_axis_name="s")

    @functools.partial(
        pl.kernel, mesh=mesh,
        compiler_params=pltpu.CompilerParams(needs_layout_passes=False),
        out_type=jax.ShapeDtypeStruct((NC, NP, D), f),
        scratch_types=[pltpu.VMEM((NCH, GB), jnp.int32),
                       [pltpu.VMEM((GB, D), f)] * 2,
                       pltpu.VMEM((32, D), f),
                       pltpu.VMEM_SHARED((NP, D), f),
                       [pltpu.SemaphoreType.DMA] * 2,
                       [pltpu.SemaphoreType.DMA] * 2],
    )
    def k(c_hbm, hi_hbm, out_hbm, hi_v, cbuf, zbuf, acc, sr, ss):
        cid = lax.axis_index("c")
        sid = lax.axis_index("s")
        wid = sid * NC + cid
        pltpu.sync_copy(hi_hbm.at[wid], hi_v)

        zero = jnp.zeros((16,), f)

        def zrow(q, carry):
            zbuf[q // 8, pl.ds((q % 8) * 16, 16)] = zero
            return carry

        lax.fori_loop(0, 32 * 8, zrow, 0)

        def zcopy(r, carry):
            pltpu.sync_copy(zbuf, acc.at[pl.ds(sid * SR + r * 32, 32)])
            return carry

        lax.fori_loop(0, SR // 32, zcopy, 0)
        plsc.subcore_barrier()

        def read_start(j, b):
            pltpu.async_copy(c_hbm.at[pl.ds(wid * EW + j * GB, GB)],
                             cbuf[b], sr[b])

        def process(j, b):
            pltpu.make_async_copy(c_hbm.at[pl.ds(0, GB)], cbuf[b],
                                  sr[b]).wait()
            pltpu.async_copy(cbuf[b], acc.at[hi_v.at[j]], ss[b], add=True)
            pltpu.make_async_copy(cbuf[b], acc.at[pl.ds(0, GB)],
                                  ss[b]).wait()

            @pl.when(j + 2 < NCH)
            def _refill():
                read_start(j + 2, b)

        read_start(0, 0)
        read_start(1, 1)

        def pair(i, carry):
            process(2 * i, 0)
            process(2 * i + 1, 1)
            return carry

        lax.fori_loop(0, NCH // 2, pair, 0)
        process(NCH - 1, 0)
        plsc.subcore_barrier()
        pltpu.sync_copy(acc.at[pl.ds(sid * SR, SR)],
                        out_hbm.at[cid, pl.ds(sid * SR, SR)])

    return k(c_arr, hi2d)


def kernel(agts, ctx, agt_ctrs, ctx_ctrs, hi, wi, dist_w1, dist_b1, dist_w2,
           dist_g2, dist_b2, query_w, query_g, query_b, ctx_w1, ctx_g1,
           ctx_b1, ctx_w2, agt_w, norm_g, norm_b, lin_w, lin_g, lin_b):
    r = lambda v: v.reshape(1, D)
    ctr = lambda w: w - jnp.mean(w, axis=1, keepdims=True)
    query_w = ctr(query_w)
    ctx_w1 = ctr(ctx_w1)
    dist_w2 = ctr(dist_w2)
    agt_w = ctr(agt_w)
    ctx_w2 = ctr(ctx_w2)
    lin_w = ctr(lin_w)
    w1d, w1q, w1c = ctx_w1[:D], ctx_w1[D:2 * D], ctx_w1[2 * D:]

    qw, cc, a0 = _node_stage(agts, ctx, query_w, r(query_g), r(query_b),
                             w1q, w1c, agt_w)

    w1p = jnp.pad(dist_w1, ((0, 6), (0, 0)))

    hi1d = hi.reshape(NW, EW)
    wi1d = wi.reshape(NW, EW)
    hi2d = hi.reshape(NW, NCH, GB)

    g, dt = _sc_gather(qw, cc, agt_ctrs[:, 0], agt_ctrs[:, 1],
                       ctx_ctrs[:, 0], ctx_ctrs[:, 1], hi1d, wi1d)

    c = _edge_stage(g, dt, w1p, r(dist_b1), dist_w2, r(dist_g2),
                    r(dist_b2), w1d, r(ctx_g1), r(ctx_b1), ctx_w2)

    part = _sc_scatter(c, hi2d)

    return _epilogue(a0, part, agts, lin_w,
                     r(norm_g), r(norm_b), r(lin_g), r(lin_b))


# speedup vs baseline: 1.0735x; 1.0735x over previous
import functools

import jax
import jax.numpy as jnp
from jax import lax
from jax.experimental import pallas as pl
from jax.experimental.pallas import tpu as pltpu
from jax.experimental.pallas import tpu_sc as plsc

N_NODES = 10000
N_EDGES = 320000
D = 128

BN = 1000
BE = 2000

NC = 2
NS = 16
NW = NC * NS
EW = N_EDGES // NW
GB = 80
NCH = EW // GB
NP = 10240
SR = NP // NS
CPB = BE // GB

NSEG = 5
ES = N_EDGES // NSEG
EWS = ES // NW
NCHS = EWS // GB
SBLK = ES // BE


def _gn0(x, g, b):
    v = jnp.mean(x * x, axis=-1, keepdims=True)
    return x * lax.rsqrt(v + 1e-5) * g + b


def _gnm(x, g, b):
    m = jnp.mean(x, axis=-1, keepdims=True)
    xc = x - m
    v = jnp.mean(xc * xc, axis=-1, keepdims=True)
    return xc * lax.rsqrt(v + 1e-5) * g + b


def _node_body(agts_ref, ctx_ref, qw_ref, qg_ref, qb_ref, w1q_ref, w1c_ref,
               agtw_ref, qwo_ref, cco_ref, a0_ref):
    agts = agts_ref[...]
    q = jnp.maximum(_gn0(jnp.dot(agts, qw_ref[...],
                                preferred_element_type=jnp.float32),
                        qg_ref[...], qb_ref[...]), 0.0)
    qwo_ref[...] = jnp.dot(q, w1q_ref[...], preferred_element_type=jnp.float32)
    cco_ref[...] = jnp.dot(ctx_ref[...], w1c_ref[...],
                           preferred_element_type=jnp.float32)
    a0_ref[...] = jnp.dot(agts, agtw_ref[...],
                          preferred_element_type=jnp.float32)


def _edge_body(g_ref, dt_ref, w1p_ref, b1_ref, dw2_ref,
               g2_ref, b2_ref, w1d_ref, g1_ref, b1c_ref, cw2_ref, c_ref):
    d = jnp.maximum(
        lax.dot_general(dt_ref[0], w1p_ref[...], (((0,), (0,)), ((), ())),
                        preferred_element_type=jnp.float32) + b1_ref[...], 0.0)
    t = jnp.maximum(
        _gn0(jnp.dot(d, dw2_ref[...], preferred_element_type=jnp.float32),
            g2_ref[...], b2_ref[...]), 0.0)
    e = jnp.dot(t, w1d_ref[...], preferred_element_type=jnp.float32)
    e = e + g_ref[...]
    e = jnp.maximum(_gn0(e, g1_ref[...], b1c_ref[...]), 0.0)
    c_ref[...] = jnp.dot(e, cw2_ref[...], preferred_element_type=jnp.float32)


def _epi_body(a0_ref, p0_ref, p1_ref, res_ref, linw_ref, ng_ref, nb_ref,
              lg_ref, lb_ref, out_ref):
    a = a0_ref[...] + p0_ref[0] + p1_ref[0]
    a = jnp.maximum(_gnm(a, ng_ref[...], nb_ref[...]), 0.0)
    a = _gn0(jnp.dot(a, linw_ref[...], preferred_element_type=jnp.float32),
            lg_ref[...], lb_ref[...])
    out_ref[...] = jnp.maximum(a + res_ref[...], 0.0)


def _full(shape):
    return pl.BlockSpec(shape, lambda i: (0,) * len(shape))


def _rows(bs, w):
    return pl.BlockSpec((bs, w), lambda i: (i, 0))


def _node_stage(agts, ctx, query_w, query_g, query_b, w1q, w1c, agt_w):
    f = jax.ShapeDtypeStruct((N_NODES, D), jnp.float32)
    return pl.pallas_call(
        _node_body,
        grid=(N_NODES // BN,),
        in_specs=[_rows(BN, D), _rows(BN, D), _full((D, D)), _full((1, D)),
                  _full((1, D)), _full((D, D)), _full((D, D)), _full((D, D))],
        out_specs=[_rows(BN, D)] * 3,
        out_shape=[f, f, f],
    )(agts, ctx, query_w, query_g, query_b, w1q, w1c, agt_w)


def _edge_stage(g, dt, w1p, b1, dw2, g2, b2, w1d, g1, b1c, cw2):
    return pl.pallas_call(
        _edge_body,
        grid=(SBLK,),
        in_specs=[_rows(BE, D),
                  pl.BlockSpec((1, 8, BE), lambda i: (i, 0, 0)),
                  _full((8, D)), _full((1, D)), _full((D, D)), _full((1, D)),
                  _full((1, D)), _full((D, D)), _full((1, D)), _full((1, D)),
                  _full((D, D))],
        out_specs=_rows(BE, D),
        out_shape=jax.ShapeDtypeStruct((ES, D), jnp.float32),
    )(g, dt, w1p, b1, dw2, g2, b2, w1d, g1, b1c, cw2)


def _epilogue(a0, part, res, lin_w, ng, nb, lg, lb):
    return pl.pallas_call(
        _epi_body,
        grid=(N_NODES // BN,),
        in_specs=[_rows(BN, D),
                  pl.BlockSpec((1, BN, D), lambda i: (0, i, 0)),
                  pl.BlockSpec((1, BN, D), lambda i: (1, i, 0)),
                  _rows(BN, D), _full((D, D))] + [_full((1, D))] * 4,
        out_specs=_rows(BN, D),
        out_shape=jax.ShapeDtypeStruct((N_NODES, D), jnp.float32),
    )(a0, part, part, res, lin_w, ng, nb, lg, lb)


def _sc_gather(qw_tab, cc_tab, ax, ay, cx, cy, hi2d, wi2d):
    f = jnp.float32
    mesh = plsc.VectorSubcoreMesh(core_axis_name="c", subcore_axis_name="s")

    @functools.partial(
        pl.kernel, mesh=mesh,
        compiler_params=pltpu.CompilerParams(needs_layout_passes=False),
        out_type=[jax.ShapeDtypeStruct((ES, D), f),
                  jax.ShapeDtypeStruct((NW, 8, BE), f)],
        scratch_types=[pltpu.VMEM((EWS,), jnp.int32),
                       pltpu.VMEM((EWS,), jnp.int32),
                       pltpu.VMEM((N_NODES,), f), pltpu.VMEM((N_NODES,), f),
                       pltpu.VMEM((N_NODES,), f), pltpu.VMEM((N_NODES,), f),
                       [pltpu.VMEM((GB, D), f)] * 2,
                       [pltpu.VMEM((GB, D), f)] * 2,
                       pltpu.VMEM((8, BE), f),
                       [pltpu.SemaphoreType.DMA] * 2,
                       [pltpu.SemaphoreType.DMA] * 2,
                       [pltpu.SemaphoreType.DMA] * 2],
    )
    def k(qw_hbm, cc_hbm, ax_hbm, ay_hbm, cx_hbm, cy_hbm, hi_hbm, wi_hbm,
          g_hbm, dt_hbm,
          hi_v, wi_v, ax_v, ay_v, cx_v, cy_v, bq, bc, dtb, sq, sc_, sw):
        wid = lax.axis_index("s") * NC + lax.axis_index("c")
        base = wid * EWS
        pltpu.sync_copy(hi_hbm.at[wid], hi_v)
        pltpu.sync_copy(wi_hbm.at[wid], wi_v)
        pltpu.sync_copy(ax_hbm, ax_v)
        pltpu.sync_copy(ay_hbm, ay_v)
        pltpu.sync_copy(cx_hbm, cx_v)
        pltpu.sync_copy(cy_hbm, cy_v)

        zero = jnp.zeros((16,), f)

        def zrow(q, carry):
            dtb[2 + q // (BE // 16), pl.ds((q % (BE // 16)) * 16, 16)] = zero
            return carry

        lax.fori_loop(0, 6 * (BE // 16), zrow, 0)

        def gather_start(j, b):
            pltpu.async_copy(qw_hbm.at[hi_v.at[pl.ds(j * GB, GB)]],
                             bq[b], sq[b])
            pltpu.async_copy(cc_hbm.at[wi_v.at[pl.ds(j * GB, GB)]],
                             bc[b], sc_[b])

        def process(j, b):
            pltpu.make_async_copy(qw_hbm.at[pl.ds(0, GB)], bq[b],
                                  sq[b]).wait()
            pltpu.make_async_copy(cc_hbm.at[pl.ds(0, GB)], bc[b],
                                  sc_[b]).wait()

            def addrow(rr, carry):
                for gi in range(D // 16):
                    s = pl.ds(gi * 16, 16)
                    bq[b][rr, s] = bq[b][rr, s] + bc[b][rr, s]
                return carry

            lax.fori_loop(0, GB, addrow, 0)

            off = (j % CPB) * GB
            for g in range(GB // 16):
                h = hi_v[pl.ds(j * GB + g * 16, 16)]
                w = wi_v[pl.ds(j * GB + g * 16, 16)]
                dtb[0, pl.ds(off + g * 16, 16)] = (
                    plsc.load_gather(ax_v, [h]) - plsc.load_gather(cx_v, [w]))
                dtb[1, pl.ds(off + g * 16, 16)] = (
                    plsc.load_gather(ay_v, [h]) - plsc.load_gather(cy_v, [w]))

            pltpu.async_copy(bq[b], g_hbm.at[pl.ds(base + j * GB, GB)], sw[b])
            pltpu.make_async_copy(bq[b], g_hbm.at[pl.ds(0, GB)], sw[b]).wait()

            @pl.when(j + 2 < NCHS)
            def _refill():
                gather_start(j + 2, b)

            @pl.when(j % CPB == CPB - 1)
            def _flush():
                pltpu.sync_copy(dtb, dt_hbm.at[wid])

        gather_start(0, 0)
        gather_start(1, 1)

        def pair(i, carry):
            process(2 * i, 0)
            process(2 * i + 1, 1)
            return carry

        lax.fori_loop(0, NCHS // 2, pair, 0)
        process(NCHS - 1, 0)

    return k(qw_tab, cc_tab, ax, ay, cx, cy, hi2d, wi2d)


def _sc_scatter(c_arr, hi2d):
    f = jnp.float32
    mesh = plsc.VectorSubcoreMesh(core_axis_name="c", subcore_axis_name="s")

    @functools.partial(
        pl.kernel, mesh=mesh,
        compiler_params=pltpu.CompilerParams(needs_layout_passes=False),
        out_type=jax.ShapeDtypeStruct((NC, NP, D), f),
        scratch_types=[pltpu.VMEM((NCH, GB), jnp.int32),
                       [pltpu.VMEM((GB, D), f)] * 2,
                       pltpu.VMEM((32, D), f),
                       pltpu.VMEM_SHARED((NP, D), f),
                       [pltpu.SemaphoreType.DMA] * 2,
                       [pltpu.SemaphoreType.DMA] * 2],
    )
    def k(c0, c1, c2, c3, c4, hi_hbm, out_hbm, hi_v, cbuf, zbuf, acc, sr, ss):
        segs = (c0, c1, c2, c3, c4)
        cid = lax.axis_index("c")
        sid = lax.axis_index("s")
        wid = sid * NC + cid
        pltpu.sync_copy(hi_hbm.at[wid], hi_v)

        zero = jnp.zeros((16,), f)

        def zrow(q, carry):
            zbuf[q // 8, pl.ds((q % 8) * 16, 16)] = zero
            return carry

        lax.fori_loop(0, 32 * 8, zrow, 0)

        def zcopy(r, carry):
            pltpu.sync_copy(zbuf, acc.at[pl.ds(sid * SR + r * 32, 32)])
            return carry

        lax.fori_loop(0, SR // 32, zcopy, 0)
        plsc.subcore_barrier()

        def read_start(s, lj, b):
            pltpu.async_copy(segs[s].at[pl.ds(wid * EWS + lj * GB, GB)],
                             cbuf[b], sr[b])

        def process(s, lj, b):
            pltpu.make_async_copy(segs[s].at[pl.ds(0, GB)], cbuf[b],
                                  sr[b]).wait()
            pltpu.async_copy(cbuf[b], acc.at[hi_v.at[s * NCHS + lj]],
                             ss[b], add=True)
            pltpu.make_async_copy(cbuf[b], acc.at[pl.ds(0, GB)],
                                  ss[b]).wait()
            if lj + 2 < NCHS:
                read_start(s, lj + 2, b)

        for s in range(NSEG):
            read_start(s, 0, 0)
            read_start(s, 1, 1)

            def pair(i, carry, s=s):
                process_t(s, 2 * i, 0)
                process_t(s, 2 * i + 1, 1)
                return carry

            def process_t(s2, lj, b):
                pltpu.make_async_copy(segs[s2].at[pl.ds(0, GB)], cbuf[b],
                                      sr[b]).wait()
                pltpu.async_copy(cbuf[b], acc.at[hi_v.at[s2 * NCHS + lj]],
                                 ss[b], add=True)
                pltpu.make_async_copy(cbuf[b], acc.at[pl.ds(0, GB)],
                                      ss[b]).wait()

                @pl.when(lj + 2 < NCHS)
                def _refill():
                    pltpu.async_copy(
                        segs[s2].at[pl.ds(wid * EWS + (lj + 2) * GB, GB)],
                        cbuf[b], sr[b])

            lax.fori_loop(0, NCHS // 2, pair, 0)
            process(s, NCHS - 1, 0)

        plsc.subcore_barrier()
        pltpu.sync_copy(acc.at[pl.ds(sid * SR, SR)],
                        out_hbm.at[cid, pl.ds(sid * SR, SR)])

    return k(*c_arr, hi2d)


def kernel(agts, ctx, agt_ctrs, ctx_ctrs, hi, wi, dist_w1, dist_b1, dist_w2,
           dist_g2, dist_b2, query_w, query_g, query_b, ctx_w1, ctx_g1,
           ctx_b1, ctx_w2, agt_w, norm_g, norm_b, lin_w, lin_g, lin_b):
    r = lambda v: v.reshape(1, D)
    ctr = lambda w: w - jnp.mean(w, axis=1, keepdims=True)
    query_w = ctr(query_w)
    ctx_w1 = ctr(ctx_w1)
    dist_w2 = ctr(dist_w2)
    agt_w = ctr(agt_w)
    ctx_w2 = ctr(ctx_w2)
    lin_w = ctr(lin_w)
    w1d, w1q, w1c = ctx_w1[:D], ctx_w1[D:2 * D], ctx_w1[2 * D:]

    qw, cc, a0 = _node_stage(agts, ctx, query_w, r(query_g), r(query_b),
                             w1q, w1c, agt_w)

    w1p = jnp.pad(dist_w1, ((0, 6), (0, 0)))

    hi_seg = hi.reshape(NSEG, NW, EWS)
    wi_seg = wi.reshape(NSEG, NW, EWS)
    hi_sc = hi.reshape(NSEG, NW, NCHS, GB).transpose(1, 0, 2, 3)
    hi_sc = hi_sc.reshape(NW, NCH, GB)

    ax, ay = agt_ctrs[:, 0], agt_ctrs[:, 1]
    cx, cy = ctx_ctrs[:, 0], ctx_ctrs[:, 1]

    cs = []
    for s in range(NSEG):
        g, dt = _sc_gather(qw, cc, ax, ay, cx, cy, hi_seg[s], wi_seg[s])
        cs.append(_edge_stage(g, dt, w1p, r(dist_b1), dist_w2, r(dist_g2),
                              r(dist_b2), w1d, r(ctx_g1), r(ctx_b1), ctx_w2))

    part = _sc_scatter(cs, hi_sc)

    return _epilogue(a0, part, agts, lin_w,
                     r(norm_g), r(norm_b), r(lin_g), r(lin_b))


# speedup vs baseline: 1.2229x; 1.1392x over previous
import functools

import jax
import jax.numpy as jnp
from jax import lax
from jax.experimental import pallas as pl
from jax.experimental.pallas import tpu as pltpu
from jax.experimental.pallas import tpu_sc as plsc

N_NODES = 10000
N_EDGES = 320000
D = 128

BN = 1000
BE = 2000

NC = 2
NS = 16
NW = NC * NS
EW = N_EDGES // NW
GB = 80
NCH = EW // GB
NP = 10240
SR = NP // NS
CPB = BE // GB

NSEG = 5
ES = N_EDGES // NSEG
EWS = ES // NW
NCHS = EWS // GB
SBLK = ES // BE


def _gn0(x, g, b):
    v = jnp.mean(x * x, axis=-1, keepdims=True)
    return x * lax.rsqrt(v + 1e-5) * g + b


def _gnm(x, g, b):
    m = jnp.mean(x, axis=-1, keepdims=True)
    xc = x - m
    v = jnp.mean(xc * xc, axis=-1, keepdims=True)
    return xc * lax.rsqrt(v + 1e-5) * g + b


def _node_body(agts_ref, ctx_ref, qw_ref, qg_ref, qb_ref, w1q_ref, w1c_ref,
               agtw_ref, qwo_ref, cco_ref, a0_ref):
    agts = agts_ref[...]
    q = jnp.maximum(_gn0(jnp.dot(agts, qw_ref[...],
                                preferred_element_type=jnp.float32),
                        qg_ref[...], qb_ref[...]), 0.0)
    qwo_ref[...] = jnp.dot(q, w1q_ref[...], preferred_element_type=jnp.float32)
    cco_ref[...] = jnp.dot(ctx_ref[...], w1c_ref[...],
                           preferred_element_type=jnp.float32)
    a0_ref[...] = jnp.dot(agts, agtw_ref[...],
                          preferred_element_type=jnp.float32)


def _edge_body(g_ref, dt_ref, w1p_ref, b1_ref, dw2_ref,
               g2_ref, b2_ref, w1d_ref, g1_ref, b1c_ref, cw2_ref, c_ref):
    d = jnp.maximum(
        lax.dot_general(dt_ref[0], w1p_ref[...], (((0,), (0,)), ((), ())),
                        preferred_element_type=jnp.float32) + b1_ref[...], 0.0)
    t = jnp.maximum(
        _gn0(jnp.dot(d, dw2_ref[...], preferred_element_type=jnp.float32),
            g2_ref[...], b2_ref[...]), 0.0)
    e = jnp.dot(t, w1d_ref[...], preferred_element_type=jnp.float32)
    e = e + g_ref[...]
    e = jnp.maximum(_gn0(e, g1_ref[...], b1c_ref[...]), 0.0)
    c_ref[...] = jnp.dot(e, cw2_ref[...], preferred_element_type=jnp.float32)


def _epi_body(a0_ref, p0_ref, p1_ref, p2_ref, p3_ref, res_ref, linw_ref,
              ng_ref, nb_ref, lg_ref, lb_ref, out_ref):
    a = a0_ref[...] + (p0_ref[0] + p1_ref[0]) + (p2_ref[0] + p3_ref[0])
    a = jnp.maximum(_gnm(a, ng_ref[...], nb_ref[...]), 0.0)
    a = _gn0(jnp.dot(a, linw_ref[...], preferred_element_type=jnp.float32),
            lg_ref[...], lb_ref[...])
    out_ref[...] = jnp.maximum(a + res_ref[...], 0.0)


def _full(shape):
    return pl.BlockSpec(shape, lambda i: (0,) * len(shape))


def _rows(bs, w):
    return pl.BlockSpec((bs, w), lambda i: (i, 0))


def _node_stage(agts, ctx, query_w, query_g, query_b, w1q, w1c, agt_w):
    f = jax.ShapeDtypeStruct((N_NODES, D), jnp.float32)
    return pl.pallas_call(
        _node_body,
        grid=(N_NODES // BN,),
        in_specs=[_rows(BN, D), _rows(BN, D), _full((D, D)), _full((1, D)),
                  _full((1, D)), _full((D, D)), _full((D, D)), _full((D, D))],
        out_specs=[_rows(BN, D)] * 3,
        out_shape=[f, f, f],
    )(agts, ctx, query_w, query_g, query_b, w1q, w1c, agt_w)


def _edge_stage(g, dt, w1p, b1, dw2, g2, b2, w1d, g1, b1c, cw2, dt_map):
    return pl.pallas_call(
        _edge_body,
        grid=(SBLK,),
        in_specs=[_rows(BE, D),
                  pl.BlockSpec((1, 8, BE), dt_map),
                  _full((8, D)), _full((1, D)), _full((D, D)), _full((1, D)),
                  _full((1, D)), _full((D, D)), _full((1, D)), _full((1, D)),
                  _full((D, D))],
        out_specs=_rows(BE, D),
        out_shape=jax.ShapeDtypeStruct((ES, D), jnp.float32),
    )(g, dt, w1p, b1, dw2, g2, b2, w1d, g1, b1c, cw2)


def _epilogue(a0, pa, pb, res, lin_w, ng, nb, lg, lb):
    return pl.pallas_call(
        _epi_body,
        grid=(N_NODES // BN,),
        in_specs=[_rows(BN, D),
                  pl.BlockSpec((1, BN, D), lambda i: (0, i, 0)),
                  pl.BlockSpec((1, BN, D), lambda i: (1, i, 0)),
                  pl.BlockSpec((1, BN, D), lambda i: (0, i, 0)),
                  pl.BlockSpec((1, BN, D), lambda i: (1, i, 0)),
                  _rows(BN, D), _full((D, D))] + [_full((1, D))] * 4,
        out_specs=_rows(BN, D),
        out_shape=jax.ShapeDtypeStruct((N_NODES, D), jnp.float32),
    )(a0, pa, pa, pb, pb, res, lin_w, ng, nb, lg, lb)


def _sc_dist(ax, ay, cx, cy, hi_dt, wi_dt):
    f = jnp.float32
    mesh = plsc.VectorSubcoreMesh(core_axis_name="c", subcore_axis_name="s")

    @functools.partial(
        pl.kernel, mesh=mesh,
        compiler_params=pltpu.CompilerParams(needs_layout_passes=False),
        out_type=jax.ShapeDtypeStruct((NSEG * NW, 8, BE), f),
        scratch_types=[pltpu.VMEM((EW,), jnp.int32),
                       pltpu.VMEM((EW,), jnp.int32),
                       pltpu.VMEM((N_NODES,), f), pltpu.VMEM((N_NODES,), f),
                       pltpu.VMEM((N_NODES,), f), pltpu.VMEM((N_NODES,), f),
                       pltpu.VMEM((8, BE), f)],
    )
    def k(ax_hbm, ay_hbm, cx_hbm, cy_hbm, hi_hbm, wi_hbm, dt_hbm,
          hi_v, wi_v, ax_v, ay_v, cx_v, cy_v, dtb):
        wid = lax.axis_index("s") * NC + lax.axis_index("c")
        pltpu.sync_copy(hi_hbm.at[wid], hi_v)
        pltpu.sync_copy(wi_hbm.at[wid], wi_v)
        pltpu.sync_copy(ax_hbm, ax_v)
        pltpu.sync_copy(ay_hbm, ay_v)
        pltpu.sync_copy(cx_hbm, cx_v)
        pltpu.sync_copy(cy_hbm, cy_v)

        zero = jnp.zeros((16,), f)

        def zrow(q, carry):
            dtb[2 + q // (BE // 16), pl.ds((q % (BE // 16)) * 16, 16)] = zero
            return carry

        lax.fori_loop(0, 6 * (BE // 16), zrow, 0)

        def body(j, carry):
            off = (j % CPB) * GB
            for g in range(GB // 16):
                h = hi_v[pl.ds(j * GB + g * 16, 16)]
                w = wi_v[pl.ds(j * GB + g * 16, 16)]
                dtb[0, pl.ds(off + g * 16, 16)] = (
                    plsc.load_gather(ax_v, [h]) - plsc.load_gather(cx_v, [w]))
                dtb[1, pl.ds(off + g * 16, 16)] = (
                    plsc.load_gather(ay_v, [h]) - plsc.load_gather(cy_v, [w]))

            @pl.when(j % CPB == CPB - 1)
            def _flush():
                pltpu.sync_copy(dtb, dt_hbm.at[(j // CPB) * NW + wid])

            return carry

        lax.fori_loop(0, NCH, body, 0)

    return k(ax, ay, cx, cy, hi_dt, wi_dt)


def _sc_gather(qw_tab, cc_tab, hi2d, wi2d):
    f = jnp.float32
    mesh = plsc.VectorSubcoreMesh(core_axis_name="c", subcore_axis_name="s")

    @functools.partial(
        pl.kernel, mesh=mesh,
        compiler_params=pltpu.CompilerParams(needs_layout_passes=False),
        out_type=jax.ShapeDtypeStruct((ES, D), f),
        scratch_types=[pltpu.VMEM((EWS,), jnp.int32),
                       pltpu.VMEM((EWS,), jnp.int32),
                       [pltpu.VMEM((GB, D), f)] * 2,
                       [pltpu.VMEM((GB, D), f)] * 2,
                       [pltpu.SemaphoreType.DMA] * 2,
                       [pltpu.SemaphoreType.DMA] * 2,
                       [pltpu.SemaphoreType.DMA] * 2],
    )
    def k(qw_hbm, cc_hbm, hi_hbm, wi_hbm, g_hbm,
          hi_v, wi_v, bq, bc, sq, sc_, sw):
        wid = lax.axis_index("s") * NC + lax.axis_index("c")
        base = wid * EWS
        pltpu.sync_copy(hi_hbm.at[wid], hi_v)
        pltpu.sync_copy(wi_hbm.at[wid], wi_v)

        def gather_start(j, b):
            pltpu.async_copy(qw_hbm.at[hi_v.at[pl.ds(j * GB, GB)]],
                             bq[b], sq[b])
            pltpu.async_copy(cc_hbm.at[wi_v.at[pl.ds(j * GB, GB)]],
                             bc[b], sc_[b])

        def process(j, b):
            pltpu.make_async_copy(qw_hbm.at[pl.ds(0, GB)], bq[b],
                                  sq[b]).wait()
            pltpu.make_async_copy(cc_hbm.at[pl.ds(0, GB)], bc[b],
                                  sc_[b]).wait()

            def addrow(rr, carry):
                for gi in range(D // 16):
                    s = pl.ds(gi * 16, 16)
                    bq[b][rr, s] = bq[b][rr, s] + bc[b][rr, s]
                return carry

            lax.fori_loop(0, GB, addrow, 0)

            pltpu.async_copy(bq[b], g_hbm.at[pl.ds(base + j * GB, GB)], sw[b])
            pltpu.make_async_copy(bq[b], g_hbm.at[pl.ds(0, GB)], sw[b]).wait()

            @pl.when(j + 2 < NCHS)
            def _refill():
                gather_start(j + 2, b)

        gather_start(0, 0)
        gather_start(1, 1)

        def pair(i, carry):
            process(2 * i, 0)
            process(2 * i + 1, 1)
            return carry

        lax.fori_loop(0, NCHS // 2, pair, 0)
        process(NCHS - 1, 0)

    return k(qw_tab, cc_tab, hi2d, wi2d)


def _sc_scatter(c_arr, hi2d):
    f = jnp.float32
    ns = len(c_arr)
    mesh = plsc.VectorSubcoreMesh(core_axis_name="c", subcore_axis_name="s")

    @functools.partial(
        pl.kernel, mesh=mesh,
        compiler_params=pltpu.CompilerParams(needs_layout_passes=False),
        out_type=jax.ShapeDtypeStruct((NC, NP, D), f),
        scratch_types=[pltpu.VMEM((ns * NCHS, GB), jnp.int32),
                       [pltpu.VMEM((GB, D), f)] * 2,
                       pltpu.VMEM((32, D), f),
                       pltpu.VMEM_SHARED((NP, D), f),
                       [pltpu.SemaphoreType.DMA] * 2,
                       [pltpu.SemaphoreType.DMA] * 2],
    )
    def k(*refs):
        segs = refs[:ns]
        hi_hbm, out_hbm, hi_v, cbuf, zbuf, acc, sr, ss = refs[ns:]
        cid = lax.axis_index("c")
        sid = lax.axis_index("s")
        wid = sid * NC + cid
        pltpu.sync_copy(hi_hbm.at[wid], hi_v)

        zero = jnp.zeros((16,), f)

        def zrow(q, carry):
            zbuf[q // 8, pl.ds((q % 8) * 16, 16)] = zero
            return carry

        lax.fori_loop(0, 32 * 8, zrow, 0)

        def zcopy(r, carry):
            pltpu.sync_copy(zbuf, acc.at[pl.ds(sid * SR + r * 32, 32)])
            return carry

        lax.fori_loop(0, SR // 32, zcopy, 0)
        plsc.subcore_barrier()

        def read_start(s, lj, b):
            pltpu.async_copy(segs[s].at[pl.ds(wid * EWS + lj * GB, GB)],
                             cbuf[b], sr[b])

        def process_t(s2, lj, b):
            pltpu.make_async_copy(segs[s2].at[pl.ds(0, GB)], cbuf[b],
                                  sr[b]).wait()
            pltpu.async_copy(cbuf[b], acc.at[hi_v.at[s2 * NCHS + lj]],
                             ss[b], add=True)
            pltpu.make_async_copy(cbuf[b], acc.at[pl.ds(0, GB)],
                                  ss[b]).wait()

            @pl.when(lj + 2 < NCHS)
            def _refill():
                pltpu.async_copy(
                    segs[s2].at[pl.ds(wid * EWS + (lj + 2) * GB, GB)],
                    cbuf[b], sr[b])

        for s in range(ns):
            read_start(s, 0, 0)
            read_start(s, 1, 1)

            def pair(i, carry, s=s):
                process_t(s, 2 * i, 0)
                process_t(s, 2 * i + 1, 1)
                return carry

            lax.fori_loop(0, NCHS // 2, pair, 0)
            process_t(s, jnp.int32(NCHS - 1), 0)

        plsc.subcore_barrier()
        pltpu.sync_copy(acc.at[pl.ds(sid * SR, SR)],
                        out_hbm.at[cid, pl.ds(sid * SR, SR)])

    return k(*c_arr, hi2d)


def kernel(agts, ctx, agt_ctrs, ctx_ctrs, hi, wi, dist_w1, dist_b1, dist_w2,
           dist_g2, dist_b2, query_w, query_g, query_b, ctx_w1, ctx_g1,
           ctx_b1, ctx_w2, agt_w, norm_g, norm_b, lin_w, lin_g, lin_b):
    r = lambda v: v.reshape(1, D)
    ctr = lambda w: w - jnp.mean(w, axis=1, keepdims=True)
    query_w = ctr(query_w)
    ctx_w1 = ctr(ctx_w1)
    dist_w2 = ctr(dist_w2)
    agt_w = ctr(agt_w)
    ctx_w2 = ctr(ctx_w2)
    lin_w = ctr(lin_w)
    w1d, w1q, w1c = ctx_w1[:D], ctx_w1[D:2 * D], ctx_w1[2 * D:]

    qw, cc, a0 = _node_stage(agts, ctx, query_w, r(query_g), r(query_b),
                             w1q, w1c, agt_w)

    w1p = jnp.pad(dist_w1, ((0, 6), (0, 0)))

    hi_seg = hi.reshape(NSEG, NW, EWS)
    wi_seg = wi.reshape(NSEG, NW, EWS)
    hi_sc = hi.reshape(NSEG, NW, NCHS, GB).transpose(1, 0, 2, 3)
    hi_sc = hi_sc.reshape(NW, NCH, GB)

    ax, ay = agt_ctrs[:, 0], agt_ctrs[:, 1]
    cx, cy = ctx_ctrs[:, 0], ctx_ctrs[:, 1]

    hi_dt = hi_sc.reshape(NW, EW)
    wi_sc = wi.reshape(NSEG, NW, NCHS, GB).transpose(1, 0, 2, 3)
    wi_dt = wi_sc.reshape(NW, EW)
    dt = _sc_dist(ax, ay, cx, cy, hi_dt, wi_dt)

    cs = []
    for s in range(NSEG):
        g = _sc_gather(qw, cc, hi_seg[s], wi_seg[s])
        dts = lambda i, s=s: (s * NW + i, 0, 0)
        cs.append(_edge_stage(g, dt, w1p, r(dist_b1), dist_w2, r(dist_g2),
                              r(dist_b2), w1d, r(ctx_g1), r(ctx_b1), ctx_w2,
                              dt_map=dts))

    part_a = _sc_scatter(cs[:3], hi_sc[:, :3 * NCHS])
    part_b = _sc_scatter(cs[3:], hi_sc[:, 3 * NCHS:])

    return _epilogue(a0, part_a, part_b, agts, lin_w,
                     r(norm_g), r(norm_b), r(lin_g), r(lin_b))
